# baseline (speedup 1.0000x reference)
import functools

import jax
import jax.numpy as jnp
from jax import lax
from jax.experimental import pallas as pl
from jax.experimental.pallas import tpu as pltpu
from jax.experimental.pallas import tpu_sc as plsc

NUM_CORES = 2
NUM_SUBCORES = 16
NUM_WORKERS = NUM_CORES * NUM_SUBCORES
BBLK = 128
NBUF = 5


def _emb_call(seq, bblocks, emb_dim):
    mesh = plsc.VectorSubcoreMesh(
        core_axis_name="c",
        subcore_axis_name="s",
        num_cores=NUM_CORES,
        num_subcores=NUM_SUBCORES,
    )
    dh = emb_dim // 8

    @functools.partial(
        pl.kernel,
        out_type=jax.ShapeDtypeStruct((seq, dh, bblocks, 8, BBLK), jnp.float32),
        mesh=mesh,
        compiler_params=pltpu.CompilerParams(
            use_tc_tiling_on_sc=False, needs_layout_passes=False
        ),
        scratch_types=[
            pltpu.VMEM((seq, BBLK), jnp.int32),
            pltpu.VMEM((NBUF, BBLK, emb_dim), jnp.float32),
            pltpu.VMEM((NBUF, dh, 8, BBLK), jnp.float32),
            [pltpu.SemaphoreType.DMA] * NBUF,
            [pltpu.SemaphoreType.DMA] * NBUF,
        ],
    )
    def emb(x_hbm, tbl_hbm, out_hbm, idx_v, rows_v, t_v, gsems, ssems):
        wid = lax.axis_index("s") * NUM_CORES + lax.axis_index("c")
        pltpu.sync_copy(x_hbm.at[wid], idx_v)

        iota = lax.iota(jnp.int32, 16)
        rots = [(iota + r) & 15 for r in range(16)]
        colv = [16 * h + iota for h in range(4)]
        dh_c = [(16 * h + iota) >> 3 for h in range(4)]
        dl_c = [(16 * h + iota) & 7 for h in range(4)]

        for b in range(NBUF):
            pltpu.async_copy(tbl_hbm.at[idx_v.at[b]], rows_v.at[b], gsems[b])

        @pl.loop(0, seq, step=NBUF)
        def _round(j):
            for b in range(NBUF):
                k = j + b
                pltpu.make_async_copy(
                    tbl_hbm.at[idx_v.at[k]], rows_v.at[b], gsems[b]
                ).wait()

                @pl.when(k >= NBUF)
                def _():
                    pltpu.make_async_copy(
                        t_v.at[b], out_hbm.at[k, :, wid, :, :], ssems[b]
                    ).wait()

                rows2 = rows_v.at[b]
                t3 = t_v.at[b]

                def step(rowv, h):
                    return rowv, plsc.load_gather(rows2, [rowv, colv[h]]), h

                def flush(pend):
                    prow, pval, ph = pend
                    plsc.store_scatter(t3, [dh_c[ph], dl_c[ph], prow], pval)

                init = step(rots[0], 0)

                @pl.loop(0, 8, init_carry=init[:2])
                def _g(g, carry):
                    g16 = jnp.broadcast_to(g * 16, (16,))
                    pend = (carry[0], carry[1], 3)
                    for h in range(4):
                        for r in range(16):
                            cur = step(rots[r] + g16, h)
                            flush(pend)
                            pend = cur
                    return pend[:2]

                flush((_g[0], _g[1], 3))

                pltpu.async_copy(t_v.at[b], out_hbm.at[k, :, wid, :, :], ssems[b])

                @pl.when(k + NBUF < seq)
                def _():
                    pltpu.async_copy(
                        tbl_hbm.at[idx_v.at[k + NBUF]], rows_v.at[b], gsems[b]
                    )

        for b in range(NBUF):
            k = seq - NBUF + b
            pltpu.make_async_copy(
                t_v.at[b], out_hbm.at[k, :, wid, :, :], ssems[b]
            ).wait()

    return emb


def kernel(x, table):
    bsz, seq = x.shape
    v, d = table.shape
    bblocks = bsz // BBLK
    xw = x.T.reshape(seq, bblocks, BBLK).transpose(1, 0, 2)
    p = _emb_call(seq, bblocks, d)(xw, table)
    return p.transpose(2, 4, 0, 1, 3).reshape(bsz, seq, d)


# speedup vs baseline: 1.3031x; 1.3031x over previous
import functools

import jax
import jax.numpy as jnp
from jax import lax
from jax.experimental import pallas as pl
from jax.experimental.pallas import tpu as pltpu
from jax.experimental.pallas import tpu_sc as plsc

NUM_CORES = 2
NUM_SUBCORES = 16
NUM_WORKERS = NUM_CORES * NUM_SUBCORES
BBLK = 128
TPAD = 129
NBUF = 5


def _emb_call(seq, bblocks, emb_dim):
    mesh = plsc.VectorSubcoreMesh(
        core_axis_name="c",
        subcore_axis_name="s",
        num_cores=NUM_CORES,
        num_subcores=NUM_SUBCORES,
    )
    dh = emb_dim // 8

    @functools.partial(
        pl.kernel,
        out_type=jax.ShapeDtypeStruct((seq, dh, bblocks, 8, BBLK), jnp.float32),
        mesh=mesh,
        compiler_params=pltpu.CompilerParams(
            use_tc_tiling_on_sc=False, needs_layout_passes=False
        ),
        scratch_types=[
            pltpu.VMEM((seq, BBLK), jnp.int32),
            pltpu.VMEM((NBUF, BBLK, emb_dim), jnp.float32),
            pltpu.VMEM((NBUF, dh, 8, TPAD), jnp.float32),
            [pltpu.SemaphoreType.DMA] * NBUF,
            [pltpu.SemaphoreType.DMA] * NBUF,
        ],
    )
    def emb(x_hbm, tbl_hbm, out_hbm, idx_v, rows_v, t_v, gsems, ssems):
        wid = lax.axis_index("s") * NUM_CORES + lax.axis_index("c")
        pltpu.sync_copy(x_hbm.at[wid], idx_v)

        iota = lax.iota(jnp.int32, 16)
        zero16 = jnp.zeros((16,), jnp.int32)
        fidx0 = [
            (((16 * h + iota) >> 3) * (8 * TPAD)) + (((16 * h + iota) & 7) * TPAD)
            for h in range(4)
        ]

        for b in range(NBUF):
            pltpu.async_copy(tbl_hbm.at[idx_v.at[b]], rows_v.at[b], gsems[b])

        @pl.loop(0, seq, step=NBUF)
        def _round(j):
            for b in range(NBUF):
                k = j + b
                src = t_v.at[b, :, :, pl.ds(0, BBLK)]
                pltpu.make_async_copy(
                    tbl_hbm.at[idx_v.at[k]], rows_v.at[b], gsems[b]
                ).wait()

                @pl.when(k >= NBUF)
                def _():
                    pltpu.make_async_copy(
                        src, out_hbm.at[k, :, wid, :, :], ssems[b]
                    ).wait()

                def load4(bl):
                    return tuple(
                        rows_v[b, bl, pl.ds(16 * h, 16)] for h in range(4)
                    )

                def store4(fidx, vals):
                    for h in range(4):
                        plsc.store_scatter(
                            t_v.at[b], [zero16, zero16, fidx[h]], vals[h]
                        )

                init = tuple(fidx0) + load4(0)

                @pl.loop(1, BBLK, init_carry=init, unroll=8)
                def _bl(bl, carry):
                    fidx, vals = carry[:4], carry[4:]
                    cur = load4(bl)
                    store4(fidx, vals)
                    return tuple(f + 1 for f in fidx) + cur

                last = _bl
                store4(last[:4], last[4:])

                pltpu.async_copy(src, out_hbm.at[k, :, wid, :, :], ssems[b])

                @pl.when(k + NBUF < seq)
                def _():
                    pltpu.async_copy(
                        tbl_hbm.at[idx_v.at[k + NBUF]], rows_v.at[b], gsems[b]
                    )

        for b in range(NBUF):
            k = seq - NBUF + b
            pltpu.make_async_copy(
                t_v.at[b, :, :, pl.ds(0, BBLK)],
                out_hbm.at[k, :, wid, :, :],
                ssems[b],
            ).wait()

    return emb


def kernel(x, table):
    bsz, seq = x.shape
    v, d = table.shape
    bblocks = bsz // BBLK
    xw = x.T.reshape(seq, bblocks, BBLK).transpose(1, 0, 2)
    p = _emb_call(seq, bblocks, d)(xw, table)
    return p.transpose(2, 4, 0, 1, 3).reshape(bsz, seq, d)
